# baseline (speedup 1.0000x reference)
import dataclasses
import functools

import jax
import jax.numpy as jnp
from jax import lax
from jax.experimental import pallas as pl
from jax.experimental.pallas import tpu as pltpu
from jax.experimental.pallas import tpu_sc as plsc

N = 320000
D = 128
NSEG = 10000
SPAD = 10016

NC = 2
NSUB = 16
NW = NC * NSUB
RPT = N // NW
BLK = 80
NBUF = 2
NBLK = RPT // BLK
STAGE = 128
LANES = 16
NJ = D // LANES


def _sc_partial_sums(values, seg_ids):
    mesh = plsc.VectorSubcoreMesh(core_axis_name="c", subcore_axis_name="s")
    cp = pltpu.CompilerParams()
    if "needs_layout_passes" in pltpu.CompilerParams.__dataclass_fields__:
        cp = dataclasses.replace(cp, needs_layout_passes=False)

    @functools.partial(
        pl.kernel,
        compiler_params=cp,
        out_type=jax.ShapeDtypeStruct((NC, NSEG, D), jnp.float32),
        mesh=mesh,
        scratch_types=[
            pltpu.VMEM((NBUF, BLK, D), jnp.float32),
            pltpu.VMEM((NBUF, BLK + 8), jnp.int32),
            pltpu.VMEM((NJ, LANES), jnp.float32),
            pltpu.VMEM((STAGE, D), jnp.float32),
            pltpu.VMEM((STAGE,), jnp.int32),
            pltpu.VMEM_SHARED((SPAD, D), jnp.float32),
            pltpu.SemaphoreType.DMA,
            pltpu.SemaphoreType.DMA,
        ],
    )
    def sc_kernel(vals_hbm, ids_hbm, out_hbm, vbuf, ibuf, snap, stage_v,
                  stage_i, acc_sh, sem0, sem1):
        cid = lax.axis_index("c")
        sid = lax.axis_index("s")
        wid = cid * NSUB + sid
        row0 = wid * RPT
        sems = (sem0, sem1)
        lane = lax.iota(jnp.int32, LANES)
        zvec = jnp.zeros((LANES,), jnp.float32)

        @pl.loop(0, STAGE)
        def _(r):
            ridx = jnp.full((LANES,), r, jnp.int32)
            for j in range(NJ):
                plsc.store_scatter(stage_v, [ridx, j * LANES + lane], zvec)

        zch = NSEG // NSUB
        z0 = sid * zch
        zoff = 0
        while zoff < zch:
            cnt = min(STAGE, zch - zoff)
            pltpu.sync_copy(stage_v.at[pl.ds(0, cnt)],
                            acc_sh.at[pl.ds(z0 + zoff, cnt)])
            zoff += cnt
        plsc.subcore_barrier()

        def start_fetch(blk, b):
            pltpu.async_copy(vals_hbm.at[pl.ds(row0 + blk * BLK, BLK)],
                             vbuf.at[b], sems[b])
            pltpu.async_copy(ids_hbm.at[pl.ds(row0 + blk * BLK, BLK)],
                             ibuf.at[b, pl.ds(8, BLK)], sems[b])

        for b in range(NBUF):
            start_fetch(b, b)
        for j in range(NJ):
            snap[j] = zvec

        def emit(k, seg, accs):
            ridx = jnp.full((LANES,), k, jnp.int32)
            for j in range(NJ):
                plsc.store_scatter(stage_v, [ridx, j * LANES + lane],
                                   accs[j] - snap[j])
                snap[j] = accs[j]
            seg = jnp.where(seg < 0, NSEG, seg)
            plsc.store_scatter(stage_i, [ridx],
                               jnp.full((LANES,), seg, jnp.int32),
                               mask=lane == 0)
            kn = k + 1

            def flush():
                pltpu.sync_copy(stage_v, acc_sh.at[stage_i], add=True)
                return jnp.int32(0)

            return lax.cond(kn == STAGE, flush, lambda: kn)

        def group_step(g, c, vb, ib):
            k, prev = c[0], c[1]
            accs = c[2:]
            idv = ib[pl.ds(8 + g * LANES, LANES)]
            idvp = ib[pl.ds(7 + g * LANES, LANES)]
            nb = plsc.all_reduce_population_count(idv != idvp)[0]

            def add_row(accs, i):
                return tuple(
                    accs[j] + vb[g * LANES + i, pl.ds(j * LANES, LANES)]
                    for j in range(NJ))

            def fast(k=k, accs=accs):
                for i in range(LANES):
                    accs = add_row(accs, i)
                return (k,) + accs

            def slow(k=k, prev=prev, accs=accs):
                for i in range(LANES):
                    seg = idv[i]
                    k = lax.cond(
                        seg != prev,
                        lambda k=k, prev=prev, accs=accs: emit(k, prev, accs),
                        lambda k=k: k)
                    prev = seg
                    accs = add_row(accs, i)
                return (k,) + accs

            res = lax.cond(nb == 0, fast, slow)
            return (res[0], idv[LANES - 1]) + tuple(res[1:])

        def process_block(blk, b, c):
            pltpu.make_async_copy(vals_hbm.at[pl.ds(0, BLK)], vbuf.at[b],
                                  sems[b]).wait()
            pltpu.make_async_copy(ids_hbm.at[pl.ds(0, BLK)],
                                  ibuf.at[b, pl.ds(8, BLK)], sems[b]).wait()
            plsc.store_scatter(ibuf.at[b], [jnp.full((LANES,), 7, jnp.int32)],
                               jnp.full((LANES,), c[1], jnp.int32),
                               mask=lane == 0)
            c = lax.fori_loop(
                0, BLK // LANES,
                lambda g, cc: group_step(g, cc, vbuf.at[b], ibuf.at[b]), c)
            nxt = blk + NBUF

            @pl.when(nxt < NBLK)
            def _():
                start_fetch(nxt, b)
            return c

        def outer(g, c):
            for b in range(NBUF):
                c = process_block(g * NBUF + b, b, c)
            return c

        carry0 = (jnp.int32(0), jnp.int32(-1)) + (zvec,) * NJ
        carry = lax.fori_loop(0, NBLK // NBUF, outer, carry0)
        if NBLK % NBUF:
            carry = process_block(jnp.int32(NBLK - 1), 0, carry)

        k = emit(carry[0], carry[1], carry[2:])
        dummy = jnp.full((LANES,), NSEG, jnp.int32)
        for j in range(STAGE // LANES):
            cur = stage_i[pl.ds(j * LANES, LANES)]
            stage_i[pl.ds(j * LANES, LANES)] = jnp.where(
                j * LANES + lane >= k, dummy, cur)
        pltpu.sync_copy(stage_v, acc_sh.at[stage_i], add=True)

        plsc.subcore_barrier()
        @pl.when(sid < NSUB - 1)
        def _():
            pltpu.sync_copy(acc_sh.at[pl.ds(sid * 624, 624)],
                            out_hbm.at[cid, pl.ds(sid * 624, 624)])

        @pl.when(sid == NSUB - 1)
        def _():
            tail = NSEG - 624 * (NSUB - 1)
            pltpu.sync_copy(acc_sh.at[pl.ds(624 * (NSUB - 1), tail)],
                            out_hbm.at[cid, pl.ds(624 * (NSUB - 1), tail)])

    return sc_kernel(values, seg_ids)


def _combine_body(p_ref, o_ref):
    o_ref[...] = p_ref[0] + p_ref[1]


def _tc_combine(partials):
    return pl.pallas_call(
        _combine_body,
        out_shape=jax.ShapeDtypeStruct((NSEG, D), jnp.float32),
    )(partials)


def kernel(values, segment_ids):
    ids = segment_ids.astype(jnp.int32)
    partials = _sc_partial_sums(values, ids)
    return _tc_combine(partials)


# speedup vs baseline: 1.5244x; 1.5244x over previous
import dataclasses
import functools

import jax
import jax.numpy as jnp
from jax import lax
from jax.experimental import pallas as pl
from jax.experimental.pallas import tpu as pltpu
from jax.experimental.pallas import tpu_sc as plsc

N = 320000
D = 128
NSEG = 10000
SPAD = 10016

NC = 2
NSUB = 16
NW = NC * NSUB
RPT = N // NW
BLK = 80
NBUF = 2
NBLK = RPT // BLK
STAGE = 128
LANES = 16
NJ = D // LANES


def _sc_partial_sums(values, seg_ids):
    mesh = plsc.VectorSubcoreMesh(core_axis_name="c", subcore_axis_name="s")
    cp = pltpu.CompilerParams()
    if "needs_layout_passes" in pltpu.CompilerParams.__dataclass_fields__:
        cp = dataclasses.replace(cp, needs_layout_passes=False)

    @functools.partial(
        pl.kernel,
        compiler_params=cp,
        out_type=jax.ShapeDtypeStruct((NC, NSEG, D), jnp.float32),
        mesh=mesh,
        scratch_types=[
            pltpu.VMEM((NBUF, BLK, D), jnp.float32),
            pltpu.VMEM((NBUF, BLK + 24), jnp.int32),
            pltpu.VMEM((BLK + 1, D), jnp.float32),
            pltpu.VMEM((NJ, LANES), jnp.float32),
            pltpu.VMEM((STAGE, D), jnp.float32),
            pltpu.VMEM((STAGE,), jnp.int32),
            pltpu.VMEM_SHARED((SPAD, D), jnp.float32),
            pltpu.SemaphoreType.DMA,
            pltpu.SemaphoreType.DMA,
        ],
    )
    def sc_kernel(vals_hbm, ids_hbm, out_hbm, vbuf, ibuf, cs, snap, stage_v,
                  stage_i, acc_sh, sem0, sem1):
        cid = lax.axis_index("c")
        sid = lax.axis_index("s")
        wid = cid * NSUB + sid
        row0 = wid * RPT
        sems = (sem0, sem1)
        lane = lax.iota(jnp.int32, LANES)
        zvec = jnp.zeros((LANES,), jnp.float32)

        @pl.loop(0, STAGE)
        def _(r):
            ridx = jnp.full((LANES,), r, jnp.int32)
            for j in range(NJ):
                plsc.store_scatter(stage_v, [ridx, j * LANES + lane], zvec)

        zch = NSEG // NSUB
        z0 = sid * zch
        zoff = 0
        while zoff < zch:
            cnt = min(STAGE, zch - zoff)
            pltpu.sync_copy(stage_v.at[pl.ds(0, cnt)],
                            acc_sh.at[pl.ds(z0 + zoff, cnt)])
            zoff += cnt
        plsc.subcore_barrier()

        def start_fetch(blk, b):
            pltpu.async_copy(vals_hbm.at[pl.ds(row0 + blk * BLK, BLK)],
                             vbuf.at[b], sems[b])
            pltpu.async_copy(ids_hbm.at[pl.ds(row0 + blk * BLK, BLK)],
                             ibuf.at[b, pl.ds(8, BLK)], sems[b])

        for b in range(NBUF):
            start_fetch(b, b)
        for j in range(NJ):
            snap[j] = zvec

        def emit_at(k, p, ib):
            ridx = jnp.full((LANES,), k, jnp.int32)
            for j in range(NJ):
                cj = cs[p, pl.ds(j * LANES, LANES)]
                plsc.store_scatter(stage_v, [ridx, j * LANES + lane],
                                   cj - snap[j])
                snap[j] = cj
            seg = ib[pl.ds(7 + p, LANES)][0]
            seg = jnp.where(seg < 0, NSEG, seg)
            plsc.store_scatter(stage_i, [ridx],
                               jnp.full((LANES,), seg, jnp.int32),
                               mask=lane == 0)
            kn = k + 1

            def flush():
                pltpu.sync_copy(stage_v, acc_sh.at[stage_i], add=True)
                return jnp.int32(0)

            return lax.cond(kn == STAGE, flush, lambda: kn)

        def group_rows(g, accs, vb):
            for i in range(LANES):
                accs = tuple(
                    accs[j] + vb[g * LANES + i, pl.ds(j * LANES, LANES)]
                    for j in range(NJ))
                for j in range(NJ):
                    cs[g * LANES + i + 1, pl.ds(j * LANES, LANES)] = accs[j]
            return accs

        def process_block(blk, b, c):
            pltpu.make_async_copy(vals_hbm.at[pl.ds(0, BLK)], vbuf.at[b],
                                  sems[b]).wait()
            pltpu.make_async_copy(ids_hbm.at[pl.ds(0, BLK)],
                                  ibuf.at[b, pl.ds(8, BLK)], sems[b]).wait()
            k = c[0]
            accs = c[1:]
            for j in range(NJ):
                cs[0, pl.ds(j * LANES, LANES)] = accs[j]
            accs = lax.fori_loop(
                0, BLK // LANES,
                lambda g, a: group_rows(g, a, vbuf.at[b]), accs)

            for g in range(BLK // LANES):
                idv = ibuf[b, pl.ds(8 + g * LANES, LANES)]
                idvp = ibuf[b, pl.ds(7 + g * LANES, LANES)]
                m = idv != idvp
                nb = plsc.all_reduce_population_count(m)[0]
                mi = jnp.where(m, jnp.int32(1), jnp.int32(0))

                def pop(_, km, g=g):
                    k, mi = km
                    f = plsc.all_reduce_ffs(mi != 0)[0]
                    k = emit_at(k, g * LANES + f, ibuf.at[b])
                    return k, jnp.where(lane != f, mi, jnp.int32(0))

                k = lax.cond(
                    nb > 0,
                    lambda k=k, mi=mi, nb=nb:
                        lax.fori_loop(0, nb, pop, (k, mi))[0],
                    lambda k=k: k)

            plsc.store_scatter(
                ibuf.at[1 - b], [jnp.full((LANES,), 7, jnp.int32)],
                plsc.load_gather(ibuf.at[b],
                                 [jnp.full((LANES,), 7 + BLK, jnp.int32)]),
                mask=lane == 0)
            nxt = blk + NBUF

            @pl.when(nxt < NBLK)
            def _():
                start_fetch(nxt, b)
            return (k,) + accs

        plsc.store_scatter(ibuf.at[0], [jnp.full((LANES,), 7, jnp.int32)],
                           jnp.full((LANES,), -1, jnp.int32), mask=lane == 0)

        def outer(g, c):
            for b in range(NBUF):
                c = process_block(g * NBUF + b, b, c)
            return c

        carry0 = (jnp.int32(0),) + (zvec,) * NJ
        carry = lax.fori_loop(0, NBLK // NBUF, outer, carry0)
        if NBLK % NBUF:
            carry = process_block(jnp.int32(NBLK - 1), 0, carry)

        k = emit_at(carry[0], BLK, ibuf.at[0])
        dummy = jnp.full((LANES,), NSEG, jnp.int32)
        for j in range(STAGE // LANES):
            cur = stage_i[pl.ds(j * LANES, LANES)]
            stage_i[pl.ds(j * LANES, LANES)] = jnp.where(
                j * LANES + lane >= k, dummy, cur)
        pltpu.sync_copy(stage_v, acc_sh.at[stage_i], add=True)

        plsc.subcore_barrier()
        @pl.when(sid < NSUB - 1)
        def _():
            pltpu.sync_copy(acc_sh.at[pl.ds(sid * 624, 624)],
                            out_hbm.at[cid, pl.ds(sid * 624, 624)])

        @pl.when(sid == NSUB - 1)
        def _():
            tail = NSEG - 624 * (NSUB - 1)
            pltpu.sync_copy(acc_sh.at[pl.ds(624 * (NSUB - 1), tail)],
                            out_hbm.at[cid, pl.ds(624 * (NSUB - 1), tail)])

    return sc_kernel(values, seg_ids)


def _combine_body(p_ref, o_ref):
    o_ref[...] = p_ref[0] + p_ref[1]


def _tc_combine(partials):
    return pl.pallas_call(
        _combine_body,
        out_shape=jax.ShapeDtypeStruct((NSEG, D), jnp.float32),
    )(partials)


def kernel(values, segment_ids):
    ids = segment_ids.astype(jnp.int32)
    partials = _sc_partial_sums(values, ids)
    return _tc_combine(partials)
